# Initial kernel scaffold; baseline (speedup 1.0000x reference)
import jax
import jax.numpy as jnp
from jax.experimental import pallas as pl


def kernel(edge_index, x, struct_x, static_x, W_st0, b_st0, W_st1, b_st1, W_sa0, b_sa0, W_sa1, b_sa1, W_n0, b_n0, W_n1, b_n1, W_m0, b_m0, W_m1, b_m1, W_f0, b_f0, W_f1, b_f1, W_out, b_out):
    raise NotImplementedError("write your pallas kernel here")


# speedup vs baseline: 3.4909x; 3.4909x over previous
import functools

import jax
import jax.numpy as jnp
from jax import lax
from jax.experimental import pallas as pl
from jax.experimental.pallas import tpu as pltpu

_N = 10000
_E = 320000
_R = 2000
_G = _N // _R


def _rowspec(k):
    return pl.BlockSpec((_R, k), lambda i: (i, 0))


def _fullspec(shape):
    nd = len(shape)
    return pl.BlockSpec(shape, lambda i: (0,) * nd)


def _l2n(a):
    n = jnp.sqrt(jnp.sum(a * a, axis=1, keepdims=True))
    return a / jnp.maximum(n, 1e-12)


def _tck1_body(x_ref, sx_ref, ax_ref, W_st0, W_sa0, W_n0, W_f0, b_f0, W_f1, b_f1,
               hs_out, ha_out, hn_out, f2_out):
    x = x_ref[...]
    hs_out[...] = jnp.dot(_l2n(sx_ref[...]), W_st0[...],
                          preferred_element_type=jnp.float32)
    ha_out[...] = jnp.dot(_l2n(ax_ref[...]), W_sa0[...],
                          preferred_element_type=jnp.float32)
    hn_out[...] = jnp.dot(x, W_n0[...], preferred_element_type=jnp.float32)
    f1 = jnp.maximum(jnp.dot(x, W_f0[...], preferred_element_type=jnp.float32)
                     + b_f0[...], 0.0)
    f2_out[...] = jnp.maximum(jnp.dot(f1, W_f1[...],
                                      preferred_element_type=jnp.float32)
                              + b_f1[...], 0.0)


def _tck1(x, struct_x, static_x, W_st0, W_sa0, W_n0, W_f0, b_f0, W_f1, b_f1):
    o64 = jax.ShapeDtypeStruct((_N, 64), jnp.float32)
    return pl.pallas_call(
        _tck1_body,
        grid=(_G,),
        in_specs=[_rowspec(128), _rowspec(64), _rowspec(9),
                  _fullspec((64, 64)), _fullspec((9, 64)), _fullspec((128, 64)),
                  _fullspec((128, 64)), _fullspec((1, 64)),
                  _fullspec((64, 64)), _fullspec((1, 64))],
        out_specs=[_rowspec(64)] * 4,
        out_shape=[o64, o64, o64, o64],
    )(x, struct_x, static_x, W_st0, W_sa0, W_n0, W_f0, b_f0, W_f1, b_f1)


def _tck2_body(d0, d1, s0, s1, hs, ha, hn, t1_out, dinv_out, binv_out, c_out):
    deg_raw = d0[...] + d1[...]
    c = s0[...] + s1[...]
    dinv = lax.rsqrt(deg_raw + 1.0)
    deg_b = deg_raw - c
    binv = jnp.where(deg_b > 0, 1.0 / jnp.maximum(deg_b, 1e-12), 0.0)
    t1_out[:, 0:64] = dinv * hs[...]
    t1_out[:, 64:128] = dinv * ha[...]
    t1_out[:, 128:192] = dinv * hn[...]
    dinv_out[...] = dinv
    binv_out[...] = binv
    c_out[...] = c


def _tck2(deg0, deg1, self0, self1, hs, ha, hn):
    o1 = jax.ShapeDtypeStruct((_N, 1), jnp.float32)
    return pl.pallas_call(
        _tck2_body,
        grid=(_G,),
        in_specs=[_rowspec(1)] * 4 + [_rowspec(64)] * 3,
        out_specs=[_rowspec(192), _rowspec(1), _rowspec(1), _rowspec(1)],
        out_shape=[jax.ShapeDtypeStruct((_N, 192), jnp.float32), o1, o1, o1],
    )(deg0, deg1, self0, self1, hs, ha, hn)


def _tck3_body(p1a, p1b, t1, xa, xb, x_ref, cc, dinv_ref, binv_ref,
               W_st1, W_sa1, W_n1, W_m0, b_st0, b_sa0, b_n0,
               t2_out, t4_out):
    dinv = dinv_ref[...]
    v = dinv * (p1a[...] + p1b[...] + t1[...])
    s1 = jnp.maximum(v[:, 0:64] + b_st0[...], 0.0)
    a1 = jnp.maximum(v[:, 64:128] + b_sa0[...], 0.0)
    n1 = jnp.maximum(v[:, 128:192] + b_n0[...], 0.0)
    mxagg = binv_ref[...] * (xa[...] + xb[...] - cc[...] * x_ref[...])
    mx = _l2n(mxagg)
    t2_out[:, 0:64] = dinv * jnp.dot(s1, W_st1[...],
                                     preferred_element_type=jnp.float32)
    t2_out[:, 64:128] = dinv * jnp.dot(a1, W_sa1[...],
                                       preferred_element_type=jnp.float32)
    t2_out[:, 128:192] = dinv * jnp.dot(n1, W_n1[...],
                                        preferred_element_type=jnp.float32)
    t4_out[...] = dinv * jnp.dot(mx, W_m0[...],
                                 preferred_element_type=jnp.float32)


def _tck3(p1a, p1b, t1, xa, xb, x, cc, dinv, binv,
          W_st1, W_sa1, W_n1, W_m0, b_st0, b_sa0, b_n0):
    return pl.pallas_call(
        _tck3_body,
        grid=(_G,),
        in_specs=[_rowspec(192), _rowspec(192), _rowspec(192),
                  _rowspec(128), _rowspec(128), _rowspec(128),
                  _rowspec(1), _rowspec(1), _rowspec(1),
                  _fullspec((64, 64)), _fullspec((64, 64)), _fullspec((64, 64)),
                  _fullspec((128, 64)),
                  _fullspec((1, 64)), _fullspec((1, 64)), _fullspec((1, 64))],
        out_specs=[_rowspec(192), _rowspec(64)],
        out_shape=[jax.ShapeDtypeStruct((_N, 192), jnp.float32),
                   jax.ShapeDtypeStruct((_N, 64), jnp.float32)],
    )(p1a, p1b, t1, xa, xb, x, cc, dinv, binv,
      W_st1, W_sa1, W_n1, W_m0, b_st0, b_sa0, b_n0)


def _tck4_body(p3a, p3b, t2, p4a, p4b, t4, dinv_ref,
               b_st1, b_sa1, b_n1, b_m0, W_m1, san_out, t5_out):
    dinv = dinv_ref[...]
    v = dinv * (p3a[...] + p3b[...] + t2[...])
    san_out[:, 0:64] = jnp.maximum(v[:, 0:64] + b_st1[...], 0.0)
    san_out[:, 64:128] = jnp.maximum(v[:, 64:128] + b_sa1[...], 0.0)
    san_out[:, 128:192] = jnp.maximum(v[:, 128:192] + b_n1[...], 0.0)
    m1 = jnp.maximum(dinv * (p4a[...] + p4b[...] + t4[...]) + b_m0[...], 0.0)
    t5_out[...] = dinv * jnp.dot(m1, W_m1[...],
                                 preferred_element_type=jnp.float32)


def _tck4(p3a, p3b, t2, p4a, p4b, t4, dinv, b_st1, b_sa1, b_n1, b_m0, W_m1):
    return pl.pallas_call(
        _tck4_body,
        grid=(_G,),
        in_specs=[_rowspec(192), _rowspec(192), _rowspec(192),
                  _rowspec(64), _rowspec(64), _rowspec(64), _rowspec(1),
                  _fullspec((1, 64)), _fullspec((1, 64)), _fullspec((1, 64)),
                  _fullspec((1, 64)), _fullspec((64, 64))],
        out_specs=[_rowspec(192), _rowspec(64)],
        out_shape=[jax.ShapeDtypeStruct((_N, 192), jnp.float32),
                   jax.ShapeDtypeStruct((_N, 64), jnp.float32)],
    )(p3a, p3b, t2, p4a, p4b, t4, dinv, b_st1, b_sa1, b_n1, b_m0, W_m1)


def _tck5_body(san, f2, p5a, p5b, t5, dinv_ref, b_m1,
               Wo_a, Wo_m, Wo_f, b_out, out_ref):
    m2 = jnp.maximum(dinv_ref[...] * (p5a[...] + p5b[...] + t5[...])
                     + b_m1[...], 0.0)
    out_ref[...] = (jnp.dot(san[...], Wo_a[...],
                            preferred_element_type=jnp.float32)
                    + jnp.dot(m2, Wo_m[...],
                              preferred_element_type=jnp.float32)
                    + jnp.dot(f2[...], Wo_f[...],
                              preferred_element_type=jnp.float32)
                    + b_out[...])


def _tck5(san, f2, p5a, p5b, t5, dinv, b_m1, Wo_a, Wo_m, Wo_f, b_out):
    return pl.pallas_call(
        _tck5_body,
        grid=(_G,),
        in_specs=[_rowspec(192), _rowspec(64), _rowspec(64), _rowspec(64),
                  _rowspec(64), _rowspec(1),
                  _fullspec((1, 64)), _fullspec((192, 40)),
                  _fullspec((64, 40)), _fullspec((64, 40)), _fullspec((1, 40))],
        out_specs=_rowspec(40),
        out_shape=jax.ShapeDtypeStruct((_N, 40), jnp.float32),
    )(san, f2, p5a, p5b, t5, dinv, b_m1, Wo_a, Wo_m, Wo_f, b_out)


def _agg(tab, row, col):
    return jax.ops.segment_sum(tab[col, :], row, num_segments=_N)


def kernel(edge_index, x, struct_x, static_x,
           W_st0, b_st0, W_st1, b_st1, W_sa0, b_sa0, W_sa1, b_sa1,
           W_n0, b_n0, W_n1, b_n1, W_m0, b_m0, W_m1, b_m1,
           W_f0, b_f0, W_f1, b_f1, W_out, b_out):
    row = edge_index[0]
    col = edge_index[1]
    r2 = lambda b: b.reshape(1, -1)

    hs, ha, hn, f2 = _tck1(x, struct_x, static_x, W_st0, W_sa0, W_n0,
                           W_f0, r2(b_f0), W_f1, r2(b_f1))

    ones_e = jnp.ones((_E,), jnp.float32)
    deg_raw = jax.ops.segment_sum(ones_e, row, num_segments=_N)
    selfc = jax.ops.segment_sum((row == col).astype(jnp.float32), row,
                                num_segments=_N)
    zc = jnp.zeros((_N,), jnp.float32)
    t1, dinv, binv, cc = _tck2(deg_raw.reshape(_N, 1), zc.reshape(_N, 1),
                               selfc.reshape(_N, 1), zc.reshape(_N, 1),
                               hs, ha, hn)

    p1 = _agg(t1, row, col)
    xagg = _agg(x, row, col)
    zero192 = jnp.zeros((_N, 192), jnp.float32)
    zero128 = jnp.zeros((_N, 128), jnp.float32)
    zero64 = jnp.zeros((_N, 64), jnp.float32)
    t2, t4 = _tck3(p1, zero192, t1, xagg, zero128, x, cc, dinv, binv,
                   W_st1, W_sa1, W_n1, W_m0, r2(b_st0), r2(b_sa0), r2(b_n0))

    p3 = _agg(t2, row, col)
    p4 = _agg(t4, row, col)
    san, t5 = _tck4(p3, zero192, t2, p4, zero64, t4, dinv,
                    r2(b_st1), r2(b_sa1), r2(b_n1), r2(b_m0), W_m1)

    p5 = _agg(t5, row, col)
    out = _tck5(san, f2, p5, zero64, t5, dinv, r2(b_m1),
                W_out[0:192, :], W_out[192:256, :], W_out[256:320, :],
                r2(b_out))
    return out


# speedup vs baseline: 9.3990x; 2.6925x over previous
import functools

import jax
import jax.numpy as jnp
from jax import lax
from jax.experimental import pallas as pl
from jax.experimental.pallas import tpu as pltpu
from jax.experimental.pallas import tpu_sc as plsc

_N = 10000
_E = 320000
_R = 2000
_G = _N // _R

_NP = 10112
_RPT = _NP // 16
_EW = _E // 32
_CH = 80
_NCH = _EW // _CH
_TRASH = 10048


def _rowspec(k):
    return pl.BlockSpec((_R, k), lambda i: (i, 0))


def _fullspec(shape):
    nd = len(shape)
    return pl.BlockSpec(shape, lambda i: (0,) * nd)


def _l2n(a):
    n = jnp.sqrt(jnp.sum(a * a, axis=1, keepdims=True))
    return a / jnp.maximum(n, 1e-12)


def _tck1_body(x_ref, sx_ref, ax_ref, W_st0, W_sa0, W_n0, W_f0, b_f0, W_f1, b_f1,
               hs_out, ha_out, hn_out, f2_out):
    x = x_ref[...]
    hs_out[...] = jnp.dot(_l2n(sx_ref[...]), W_st0[...],
                          preferred_element_type=jnp.float32)
    ha_out[...] = jnp.dot(_l2n(ax_ref[...]), W_sa0[...],
                          preferred_element_type=jnp.float32)
    hn_out[...] = jnp.dot(x, W_n0[...], preferred_element_type=jnp.float32)
    f1 = jnp.maximum(jnp.dot(x, W_f0[...], preferred_element_type=jnp.float32)
                     + b_f0[...], 0.0)
    f2_out[...] = jnp.maximum(jnp.dot(f1, W_f1[...],
                                      preferred_element_type=jnp.float32)
                              + b_f1[...], 0.0)


def _tck1(x, struct_x, static_x, W_st0, W_sa0, W_n0, W_f0, b_f0, W_f1, b_f1):
    o64 = jax.ShapeDtypeStruct((_N, 64), jnp.float32)
    return pl.pallas_call(
        _tck1_body,
        grid=(_G,),
        in_specs=[_rowspec(128), _rowspec(64), _rowspec(9),
                  _fullspec((64, 64)), _fullspec((9, 64)), _fullspec((128, 64)),
                  _fullspec((128, 64)), _fullspec((1, 64)),
                  _fullspec((64, 64)), _fullspec((1, 64))],
        out_specs=[_rowspec(64)] * 4,
        out_shape=[o64, o64, o64, o64],
    )(x, struct_x, static_x, W_st0, W_sa0, W_n0, W_f0, b_f0, W_f1, b_f1)


def _tck2_body(d0, d1, s0, s1, hs, ha, hn, t1_out, t1b_out,
               dinv_out, binv_out, c_out):
    deg_raw = d0[...] + d1[...]
    c = s0[...] + s1[...]
    dinv = lax.rsqrt(deg_raw + 1.0)
    deg_b = deg_raw - c
    binv = jnp.where(deg_b > 0, 1.0 / jnp.maximum(deg_b, 1e-12), 0.0)
    t1_out[:, 0:64] = dinv * hs[...]
    t1_out[:, 64:128] = dinv * ha[...]
    t1b_out[...] = dinv * hn[...]
    dinv_out[...] = dinv
    binv_out[...] = binv
    c_out[...] = c


def _tck2(deg0, deg1, self0, self1, hs, ha, hn):
    o1 = jax.ShapeDtypeStruct((_N, 1), jnp.float32)
    return pl.pallas_call(
        _tck2_body,
        grid=(_G,),
        in_specs=[_rowspec(1)] * 4 + [_rowspec(64)] * 3,
        out_specs=[_rowspec(128), _rowspec(64),
                   _rowspec(1), _rowspec(1), _rowspec(1)],
        out_shape=[jax.ShapeDtypeStruct((_N, 128), jnp.float32),
                   jax.ShapeDtypeStruct((_N, 64), jnp.float32), o1, o1, o1],
    )(deg0, deg1, self0, self1, hs, ha, hn)


def _tck3_body(p1a, p1b, q1a, q1b, t1, t1b, xa, xb, x_ref, cc,
               dinv_ref, binv_ref,
               W_st1, W_sa1, W_n1, W_m0, b_st0, b_sa0, b_n0,
               t2_out, t2b_out):
    dinv = dinv_ref[...]
    v = dinv * (p1a[...] + p1b[...] + t1[...])
    s1 = jnp.maximum(v[:, 0:64] + b_st0[...], 0.0)
    a1 = jnp.maximum(v[:, 64:128] + b_sa0[...], 0.0)
    n1 = jnp.maximum(dinv * (q1a[...] + q1b[...] + t1b[...]) + b_n0[...], 0.0)
    mxagg = binv_ref[...] * (xa[...] + xb[...] - cc[...] * x_ref[...])
    mx = _l2n(mxagg)
    t2_out[:, 0:64] = dinv * jnp.dot(s1, W_st1[...],
                                     preferred_element_type=jnp.float32)
    t2_out[:, 64:128] = dinv * jnp.dot(a1, W_sa1[...],
                                       preferred_element_type=jnp.float32)
    t2b_out[:, 0:64] = dinv * jnp.dot(n1, W_n1[...],
                                      preferred_element_type=jnp.float32)
    t2b_out[:, 64:128] = dinv * jnp.dot(mx, W_m0[...],
                                        preferred_element_type=jnp.float32)


def _tck3(p1a, p1b, q1a, q1b, t1, t1b, xa, xb, x, cc, dinv, binv,
          W_st1, W_sa1, W_n1, W_m0, b_st0, b_sa0, b_n0):
    return pl.pallas_call(
        _tck3_body,
        grid=(_G,),
        in_specs=[_rowspec(128), _rowspec(128), _rowspec(64), _rowspec(64),
                  _rowspec(128), _rowspec(64),
                  _rowspec(128), _rowspec(128), _rowspec(128),
                  _rowspec(1), _rowspec(1), _rowspec(1),
                  _fullspec((64, 64)), _fullspec((64, 64)), _fullspec((64, 64)),
                  _fullspec((128, 64)),
                  _fullspec((1, 64)), _fullspec((1, 64)), _fullspec((1, 64))],
        out_specs=[_rowspec(128), _rowspec(128)],
        out_shape=[jax.ShapeDtypeStruct((_N, 128), jnp.float32),
                   jax.ShapeDtypeStruct((_N, 128), jnp.float32)],
    )(p1a, p1b, q1a, q1b, t1, t1b, xa, xb, x, cc, dinv, binv,
      W_st1, W_sa1, W_n1, W_m0, b_st0, b_sa0, b_n0)


def _tck4_body(p3a, p3b, t2, p4a, p4b, t2b, dinv_ref,
               b_st1, b_sa1, b_n1, b_m0, W_m1, san_out, t5_out):
    dinv = dinv_ref[...]
    v = dinv * (p3a[...] + p3b[...] + t2[...])
    san_out[:, 0:64] = jnp.maximum(v[:, 0:64] + b_st1[...], 0.0)
    san_out[:, 64:128] = jnp.maximum(v[:, 64:128] + b_sa1[...], 0.0)
    vb = dinv * (p4a[...] + p4b[...] + t2b[...])
    san_out[:, 128:192] = jnp.maximum(vb[:, 0:64] + b_n1[...], 0.0)
    m1 = jnp.maximum(vb[:, 64:128] + b_m0[...], 0.0)
    t5_out[...] = dinv * jnp.dot(m1, W_m1[...],
                                 preferred_element_type=jnp.float32)


def _tck4(p3a, p3b, t2, p4a, p4b, t2b, dinv, b_st1, b_sa1, b_n1, b_m0, W_m1):
    return pl.pallas_call(
        _tck4_body,
        grid=(_G,),
        in_specs=[_rowspec(128), _rowspec(128), _rowspec(128),
                  _rowspec(128), _rowspec(128), _rowspec(128), _rowspec(1),
                  _fullspec((1, 64)), _fullspec((1, 64)), _fullspec((1, 64)),
                  _fullspec((1, 64)), _fullspec((64, 64))],
        out_specs=[_rowspec(192), _rowspec(64)],
        out_shape=[jax.ShapeDtypeStruct((_N, 192), jnp.float32),
                   jax.ShapeDtypeStruct((_N, 64), jnp.float32)],
    )(p3a, p3b, t2, p4a, p4b, t2b, dinv, b_st1, b_sa1, b_n1, b_m0, W_m1)


def _tck5_body(san, f2, p5a, p5b, t5, dinv_ref, b_m1,
               Wo_a, Wo_m, Wo_f, b_out, out_ref):
    m2 = jnp.maximum(dinv_ref[...] * (p5a[...] + p5b[...] + t5[...])
                     + b_m1[...], 0.0)
    out_ref[...] = (jnp.dot(san[...], Wo_a[...],
                            preferred_element_type=jnp.float32)
                    + jnp.dot(m2, Wo_m[...],
                              preferred_element_type=jnp.float32)
                    + jnp.dot(f2[...], Wo_f[...],
                              preferred_element_type=jnp.float32)
                    + b_out[...])


def _tck5(san, f2, p5a, p5b, t5, dinv, b_m1, Wo_a, Wo_m, Wo_f, b_out):
    return pl.pallas_call(
        _tck5_body,
        grid=(_G,),
        in_specs=[_rowspec(192), _rowspec(64), _rowspec(64), _rowspec(64),
                  _rowspec(64), _rowspec(1),
                  _fullspec((1, 64)), _fullspec((192, 40)),
                  _fullspec((64, 40)), _fullspec((64, 40)), _fullspec((1, 40))],
        out_specs=_rowspec(40),
        out_shape=jax.ShapeDtypeStruct((_N, 40), jnp.float32),
    )(san, f2, p5a, p5b, t5, dinv, b_m1, Wo_a, Wo_m, Wo_f, b_out)


def _zero_shared(zb, acc, sid, width):
    zv = jnp.zeros((16,), jnp.float32)
    for r in range(8):
        for c2 in range(width // 16):
            zb[r, pl.ds(c2 * 16, 16)] = zv

    def zbody(j, carry):
        pltpu.sync_copy(zb, acc.at[pl.ds(sid * _RPT + j * 8, 8)])
        return carry

    lax.fori_loop(0, _RPT // 8, zbody, 0)


def _sc_agg(width):
    mesh = plsc.VectorSubcoreMesh(core_axis_name="c", subcore_axis_name="s")

    @functools.partial(
        pl.kernel, mesh=mesh,
        out_type=jax.ShapeDtypeStruct((2, _NP, width), jnp.float32),
        compiler_params=pltpu.CompilerParams(use_tc_tiling_on_sc=False),
        scratch_types=[
            pltpu.VMEM((_CH,), jnp.int32),
            pltpu.VMEM((_CH,), jnp.int32),
            pltpu.VMEM((_CH, width), jnp.float32),
            pltpu.VMEM((8, width), jnp.float32),
            pltpu.VMEM_SHARED((_NP, width), jnp.float32),
            pltpu.SemaphoreType.DMA,
        ])
    def f(tab, rowh, colh, out, rowb, colb, gbuf, zb, acc, sem):
        cid = lax.axis_index("c")
        sid = lax.axis_index("s")
        wid = sid * 2 + cid
        _zero_shared(zb, acc, sid, width)
        plsc.subcore_barrier()

        def body(i, carry):
            base = wid * _EW + i * _CH
            pltpu.sync_copy(rowh.at[pl.ds(base, _CH)], rowb)
            pltpu.sync_copy(colh.at[pl.ds(base, _CH)], colb)
            pltpu.async_copy(tab.at[colb], gbuf, sem).wait()
            pltpu.sync_copy(gbuf, acc.at[rowb], add=True)
            return carry

        lax.fori_loop(0, _NCH, body, 0)
        plsc.subcore_barrier()
        pltpu.sync_copy(acc.at[pl.ds(sid * _RPT, _RPT)],
                        out.at[cid].at[pl.ds(sid * _RPT, _RPT)])

    return f


def _sc_agg_x_deg():
    mesh = plsc.VectorSubcoreMesh(core_axis_name="c", subcore_axis_name="s")

    @functools.partial(
        pl.kernel, mesh=mesh,
        out_type=[jax.ShapeDtypeStruct((2, _NP, 128), jnp.float32),
                  jax.ShapeDtypeStruct((2, _NP, 16), jnp.float32),
                  jax.ShapeDtypeStruct((2, _NP, 16), jnp.float32)],
        scratch_types=[
            pltpu.VMEM((_CH,), jnp.int32),
            pltpu.VMEM((_CH,), jnp.int32),
            pltpu.VMEM((_CH,), jnp.int32),
            pltpu.VMEM((_CH, 128), jnp.float32),
            pltpu.VMEM((8, 128), jnp.float32),
            pltpu.VMEM((_CH, 16), jnp.float32),
            pltpu.VMEM((8, 16), jnp.float32),
            pltpu.VMEM_SHARED((_NP, 128), jnp.float32),
            pltpu.VMEM_SHARED((_NP, 16), jnp.float32),
            pltpu.VMEM_SHARED((_NP, 16), jnp.float32),
            pltpu.SemaphoreType.DMA,
        ])
    def f(tab, rowh, colh, xout, dout, sout,
          rowb, colb, sb, gbuf, zb, onesb, z16, accx, accd, accs, sem):
        cid = lax.axis_index("c")
        sid = lax.axis_index("s")
        wid = sid * 2 + cid
        ones = jnp.ones((16,), jnp.float32)
        zv = jnp.zeros((16,), jnp.float32)
        for r in range(_CH):
            onesb[r, pl.ds(0, 16)] = ones
        for r in range(8):
            z16[r, pl.ds(0, 16)] = zv
        _zero_shared(zb, accx, sid, 128)

        def zbody16(j, carry):
            pltpu.sync_copy(z16, accd.at[pl.ds(sid * _RPT + j * 8, 8)])
            pltpu.sync_copy(z16, accs.at[pl.ds(sid * _RPT + j * 8, 8)])
            return carry

        lax.fori_loop(0, _RPT // 8, zbody16, 0)
        plsc.subcore_barrier()

        def body(i, carry):
            base = wid * _EW + i * _CH
            pltpu.sync_copy(rowh.at[pl.ds(base, _CH)], rowb)
            pltpu.sync_copy(colh.at[pl.ds(base, _CH)], colb)
            for k in range(_CH // 16):
                r16 = rowb[pl.ds(k * 16, 16)]
                c16 = colb[pl.ds(k * 16, 16)]
                sb[pl.ds(k * 16, 16)] = jnp.where(r16 == c16, r16, _TRASH)
            pltpu.async_copy(tab.at[colb], gbuf, sem).wait()
            pltpu.sync_copy(gbuf, accx.at[rowb], add=True)
            pltpu.sync_copy(onesb, accd.at[rowb], add=True)
            pltpu.sync_copy(onesb, accs.at[sb], add=True)
            return carry

        lax.fori_loop(0, _NCH, body, 0)
        plsc.subcore_barrier()
        pltpu.sync_copy(accx.at[pl.ds(sid * _RPT, _RPT)],
                        xout.at[cid].at[pl.ds(sid * _RPT, _RPT)])
        pltpu.sync_copy(accd.at[pl.ds(sid * _RPT, _RPT)],
                        dout.at[cid].at[pl.ds(sid * _RPT, _RPT)])
        pltpu.sync_copy(accs.at[pl.ds(sid * _RPT, _RPT)],
                        sout.at[cid].at[pl.ds(sid * _RPT, _RPT)])

    return f


def kernel(edge_index, x, struct_x, static_x,
           W_st0, b_st0, W_st1, b_st1, W_sa0, b_sa0, W_sa1, b_sa1,
           W_n0, b_n0, W_n1, b_n1, W_m0, b_m0, W_m1, b_m1,
           W_f0, b_f0, W_f1, b_f1, W_out, b_out):
    row = edge_index[0]
    col = edge_index[1]
    r2 = lambda b: b.reshape(1, -1)

    hs, ha, hn, f2 = _tck1(x, struct_x, static_x, W_st0, W_sa0, W_n0,
                           W_f0, r2(b_f0), W_f1, r2(b_f1))

    xparts = _sc_agg(128)(x, row, col)
    deg_raw = jax.ops.segment_sum(jnp.ones((_E,), jnp.float32), row,
                                  num_segments=_N).reshape(_N, 1)
    selfc = jax.ops.segment_sum((row == col).astype(jnp.float32), row,
                                num_segments=_N).reshape(_N, 1)
    zc = jnp.zeros((_N, 1), jnp.float32)
    t1, t1b, dinv, binv, cc = _tck2(deg_raw, zc, selfc, zc, hs, ha, hn)

    p1 = _sc_agg(128)(t1, row, col)
    q1 = _sc_agg(64)(t1b, row, col)
    t2, t2b = _tck3(p1[0, :_N], p1[1, :_N], q1[0, :_N], q1[1, :_N], t1, t1b,
                    xparts[0, :_N], xparts[1, :_N], x, cc, dinv, binv,
                    W_st1, W_sa1, W_n1, W_m0, r2(b_st0), r2(b_sa0), r2(b_n0))

    p3 = _sc_agg(128)(t2, row, col)
    p4 = _sc_agg(128)(t2b, row, col)
    san, t5 = _tck4(p3[0, :_N], p3[1, :_N], t2, p4[0, :_N], p4[1, :_N], t2b,
                    dinv, r2(b_st1), r2(b_sa1), r2(b_n1), r2(b_m0), W_m1)

    p5 = _sc_agg(64)(t5, row, col)
    out = _tck5(san, f2, p5[0, :_N], p5[1, :_N], t5, dinv, r2(b_m1),
                W_out[0:192, :], W_out[192:256, :], W_out[256:320, :],
                r2(b_out))
    return out


# speedup vs baseline: 11.8129x; 1.2568x over previous
import functools

import jax
import jax.numpy as jnp
from jax import lax
from jax.experimental import pallas as pl
from jax.experimental.pallas import tpu as pltpu
from jax.experimental.pallas import tpu_sc as plsc

_N = 10000
_E = 320000
_R = 2000
_G = _N // _R

_NP = 10112
_RPT = _NP // 16
_EW = _E // 32
_CH = 80
_NCH = _EW // _CH
_TRASH = 10048


def _rowspec(k):
    return pl.BlockSpec((_R, k), lambda i: (i, 0))


def _fullspec(shape):
    nd = len(shape)
    return pl.BlockSpec(shape, lambda i: (0,) * nd)


def _l2n(a):
    n = jnp.sqrt(jnp.sum(a * a, axis=1, keepdims=True))
    return a / jnp.maximum(n, 1e-12)


def _tck0_body(r_ref, c_ref, out_ref):
    r = r_ref[...]
    out_ref[...] = jnp.where(r == c_ref[...], _TRASH, r)


def _tck0(row, col):
    rr = row.reshape(2500, 128)
    cr = col.reshape(2500, 128)
    out = pl.pallas_call(
        _tck0_body,
        grid=(1,),
        in_specs=[pl.BlockSpec((2500, 128), lambda i: (0, 0))] * 2,
        out_specs=pl.BlockSpec((2500, 128), lambda i: (0, 0)),
        out_shape=jax.ShapeDtypeStruct((2500, 128), jnp.int32),
    )(rr, cr)
    return out.reshape(_E)


def _tck1_body(x_ref, sx_ref, ax_ref, W_st0, W_sa0, W_n0, W_f0, b_f0, W_f1, b_f1,
               hs_out, ha_out, hn_out, f2_out):
    x = x_ref[...]
    hs_out[...] = jnp.dot(_l2n(sx_ref[...]), W_st0[...],
                          preferred_element_type=jnp.float32)
    ha_out[...] = jnp.dot(_l2n(ax_ref[...]), W_sa0[...],
                          preferred_element_type=jnp.float32)
    hn_out[...] = jnp.dot(x, W_n0[...], preferred_element_type=jnp.float32)
    f1 = jnp.maximum(jnp.dot(x, W_f0[...], preferred_element_type=jnp.float32)
                     + b_f0[...], 0.0)
    f2_out[...] = jnp.maximum(jnp.dot(f1, W_f1[...],
                                      preferred_element_type=jnp.float32)
                              + b_f1[...], 0.0)


def _tck1(x, struct_x, static_x, W_st0, W_sa0, W_n0, W_f0, b_f0, W_f1, b_f1):
    o64 = jax.ShapeDtypeStruct((_N, 64), jnp.float32)
    return pl.pallas_call(
        _tck1_body,
        grid=(_G,),
        in_specs=[_rowspec(128), _rowspec(64), _rowspec(9),
                  _fullspec((64, 64)), _fullspec((9, 64)), _fullspec((128, 64)),
                  _fullspec((128, 64)), _fullspec((1, 64)),
                  _fullspec((64, 64)), _fullspec((1, 64))],
        out_specs=[_rowspec(64)] * 4,
        out_shape=[o64, o64, o64, o64],
    )(x, struct_x, static_x, W_st0, W_sa0, W_n0, W_f0, b_f0, W_f1, b_f1)


def _tck2_body(d0, d1, b0, b1, hs, ha, hn, t1_out, t1b_out,
               dinv_out, binv_out):
    deg_raw = d0[...] + d1[...]
    deg_b = b0[...] + b1[...]
    dinv = lax.rsqrt(deg_raw + 1.0)
    binv = jnp.where(deg_b > 0, 1.0 / jnp.maximum(deg_b, 1e-12), 0.0)
    t1_out[:, 0:64] = dinv * hs[...]
    t1_out[:, 64:128] = dinv * ha[...]
    t1b_out[...] = dinv * hn[...]
    dinv_out[...] = dinv
    binv_out[...] = binv


def _tck2(deg0, deg1, degb0, degb1, hs, ha, hn):
    o1 = jax.ShapeDtypeStruct((_N, 1), jnp.float32)
    return pl.pallas_call(
        _tck2_body,
        grid=(_G,),
        in_specs=[_rowspec(1)] * 4 + [_rowspec(64)] * 3,
        out_specs=[_rowspec(128), _rowspec(64), _rowspec(1), _rowspec(1)],
        out_shape=[jax.ShapeDtypeStruct((_N, 128), jnp.float32),
                   jax.ShapeDtypeStruct((_N, 64), jnp.float32), o1, o1],
    )(deg0, deg1, degb0, degb1, hs, ha, hn)


def _tck3_body(p1a, p1b, q1a, q1b, t1, t1b, xa, xb,
               dinv_ref, binv_ref,
               W_st1, W_sa1, W_n1, W_m0, b_st0, b_sa0, b_n0,
               t2_out, t2b_out):
    dinv = dinv_ref[...]
    v = dinv * (p1a[...] + p1b[...] + t1[...])
    s1 = jnp.maximum(v[:, 0:64] + b_st0[...], 0.0)
    a1 = jnp.maximum(v[:, 64:128] + b_sa0[...], 0.0)
    n1 = jnp.maximum(dinv * (q1a[...] + q1b[...] + t1b[...]) + b_n0[...], 0.0)
    mxagg = binv_ref[...] * (xa[...] + xb[...])
    mx = _l2n(mxagg)
    t2_out[:, 0:64] = dinv * jnp.dot(s1, W_st1[...],
                                     preferred_element_type=jnp.float32)
    t2_out[:, 64:128] = dinv * jnp.dot(a1, W_sa1[...],
                                       preferred_element_type=jnp.float32)
    t2b_out[:, 0:64] = dinv * jnp.dot(n1, W_n1[...],
                                      preferred_element_type=jnp.float32)
    t2b_out[:, 64:128] = dinv * jnp.dot(mx, W_m0[...],
                                        preferred_element_type=jnp.float32)


def _tck3(p1a, p1b, q1a, q1b, t1, t1b, xa, xb, dinv, binv,
          W_st1, W_sa1, W_n1, W_m0, b_st0, b_sa0, b_n0):
    return pl.pallas_call(
        _tck3_body,
        grid=(_G,),
        in_specs=[_rowspec(128), _rowspec(128), _rowspec(64), _rowspec(64),
                  _rowspec(128), _rowspec(64),
                  _rowspec(128), _rowspec(128),
                  _rowspec(1), _rowspec(1),
                  _fullspec((64, 64)), _fullspec((64, 64)), _fullspec((64, 64)),
                  _fullspec((128, 64)),
                  _fullspec((1, 64)), _fullspec((1, 64)), _fullspec((1, 64))],
        out_specs=[_rowspec(128), _rowspec(128)],
        out_shape=[jax.ShapeDtypeStruct((_N, 128), jnp.float32),
                   jax.ShapeDtypeStruct((_N, 128), jnp.float32)],
    )(p1a, p1b, q1a, q1b, t1, t1b, xa, xb, dinv, binv,
      W_st1, W_sa1, W_n1, W_m0, b_st0, b_sa0, b_n0)


def _tck4_body(p3a, p3b, t2, p4a, p4b, t2b, dinv_ref,
               b_st1, b_sa1, b_n1, b_m0, W_m1, san_out, t5_out):
    dinv = dinv_ref[...]
    v = dinv * (p3a[...] + p3b[...] + t2[...])
    san_out[:, 0:64] = jnp.maximum(v[:, 0:64] + b_st1[...], 0.0)
    san_out[:, 64:128] = jnp.maximum(v[:, 64:128] + b_sa1[...], 0.0)
    vb = dinv * (p4a[...] + p4b[...] + t2b[...])
    san_out[:, 128:192] = jnp.maximum(vb[:, 0:64] + b_n1[...], 0.0)
    m1 = jnp.maximum(vb[:, 64:128] + b_m0[...], 0.0)
    t5_out[...] = dinv * jnp.dot(m1, W_m1[...],
                                 preferred_element_type=jnp.float32)


def _tck4(p3a, p3b, t2, p4a, p4b, t2b, dinv, b_st1, b_sa1, b_n1, b_m0, W_m1):
    return pl.pallas_call(
        _tck4_body,
        grid=(_G,),
        in_specs=[_rowspec(128), _rowspec(128), _rowspec(128),
                  _rowspec(128), _rowspec(128), _rowspec(128), _rowspec(1),
                  _fullspec((1, 64)), _fullspec((1, 64)), _fullspec((1, 64)),
                  _fullspec((1, 64)), _fullspec((64, 64))],
        out_specs=[_rowspec(192), _rowspec(64)],
        out_shape=[jax.ShapeDtypeStruct((_N, 192), jnp.float32),
                   jax.ShapeDtypeStruct((_N, 64), jnp.float32)],
    )(p3a, p3b, t2, p4a, p4b, t2b, dinv, b_st1, b_sa1, b_n1, b_m0, W_m1)


def _tck5_body(san, f2, p5a, p5b, t5, dinv_ref, b_m1,
               Wo_a, Wo_m, Wo_f, b_out, out_ref):
    m2 = jnp.maximum(dinv_ref[...] * (p5a[...] + p5b[...] + t5[...])
                     + b_m1[...], 0.0)
    out_ref[...] = (jnp.dot(san[...], Wo_a[...],
                            preferred_element_type=jnp.float32)
                    + jnp.dot(m2, Wo_m[...],
                              preferred_element_type=jnp.float32)
                    + jnp.dot(f2[...], Wo_f[...],
                              preferred_element_type=jnp.float32)
                    + b_out[...])


def _tck5(san, f2, p5a, p5b, t5, dinv, b_m1, Wo_a, Wo_m, Wo_f, b_out):
    return pl.pallas_call(
        _tck5_body,
        grid=(_G,),
        in_specs=[_rowspec(192), _rowspec(64), _rowspec(64), _rowspec(64),
                  _rowspec(64), _rowspec(1),
                  _fullspec((1, 64)), _fullspec((192, 40)),
                  _fullspec((64, 40)), _fullspec((64, 40)), _fullspec((1, 40))],
        out_specs=_rowspec(40),
        out_shape=jax.ShapeDtypeStruct((_N, 40), jnp.float32),
    )(san, f2, p5a, p5b, t5, dinv, b_m1, Wo_a, Wo_m, Wo_f, b_out)


def _zero_shared(zb, acc, sid, width):
    zv = jnp.zeros((16,), jnp.float32)
    for r in range(8):
        for c2 in range(width // 16):
            zb[r, pl.ds(c2 * 16, 16)] = zv

    def zbody(j, carry):
        pltpu.sync_copy(zb, acc.at[pl.ds(sid * _RPT + j * 8, 8)])
        return carry

    lax.fori_loop(0, _RPT // 8, zbody, 0)


def _sc_agg(width):
    mesh = plsc.VectorSubcoreMesh(core_axis_name="c", subcore_axis_name="s")

    @functools.partial(
        pl.kernel, mesh=mesh,
        out_type=jax.ShapeDtypeStruct((2, _NP, width), jnp.float32),
        compiler_params=pltpu.CompilerParams(use_tc_tiling_on_sc=False),
        scratch_types=[
            pltpu.VMEM((_CH,), jnp.int32),
            pltpu.VMEM((_CH,), jnp.int32),
            pltpu.VMEM((_CH, width), jnp.float32),
            pltpu.VMEM((8, width), jnp.float32),
            pltpu.VMEM_SHARED((_NP, width), jnp.float32),
            pltpu.SemaphoreType.DMA,
        ])
    def f(tab, rowh, colh, out, rowb, colb, gbuf, zb, acc, sem):
        cid = lax.axis_index("c")
        sid = lax.axis_index("s")
        wid = sid * 2 + cid
        _zero_shared(zb, acc, sid, width)
        plsc.subcore_barrier()

        def body(i, carry):
            base = wid * _EW + i * _CH
            pltpu.sync_copy(rowh.at[pl.ds(base, _CH)], rowb)
            pltpu.sync_copy(colh.at[pl.ds(base, _CH)], colb)
            pltpu.async_copy(tab.at[colb], gbuf, sem).wait()
            pltpu.sync_copy(gbuf, acc.at[rowb], add=True)
            return carry

        lax.fori_loop(0, _NCH, body, 0)
        plsc.subcore_barrier()
        pltpu.sync_copy(acc.at[pl.ds(sid * _RPT, _RPT)],
                        out.at[cid].at[pl.ds(sid * _RPT, _RPT)])

    return f


def _sc_agg_x_deg():
    mesh = plsc.VectorSubcoreMesh(core_axis_name="c", subcore_axis_name="s")

    @functools.partial(
        pl.kernel, mesh=mesh,
        out_type=[jax.ShapeDtypeStruct((2, _NP, 128), jnp.float32),
                  jax.ShapeDtypeStruct((2, _NP, 16), jnp.float32),
                  jax.ShapeDtypeStruct((2, _NP, 16), jnp.float32)],
        compiler_params=pltpu.CompilerParams(use_tc_tiling_on_sc=False),
        scratch_types=[
            pltpu.VMEM((_CH,), jnp.int32),
            pltpu.VMEM((_CH,), jnp.int32),
            pltpu.VMEM((_CH,), jnp.int32),
            pltpu.VMEM((_CH, 128), jnp.float32),
            pltpu.VMEM((8, 128), jnp.float32),
            pltpu.VMEM((_CH, 16), jnp.float32),
            pltpu.VMEM((8, 16), jnp.float32),
            pltpu.VMEM_SHARED((_NP, 128), jnp.float32),
            pltpu.VMEM_SHARED((_NP, 16), jnp.float32),
            pltpu.VMEM_SHARED((_NP, 16), jnp.float32),
            pltpu.SemaphoreType.DMA,
        ])
    def f(tab, rowh, row2h, colh, xout, dout, bout,
          rowb, rb2, colb, gbuf, zb, onesb, z16, accx, accd, accb, sem):
        cid = lax.axis_index("c")
        sid = lax.axis_index("s")
        wid = sid * 2 + cid
        ones = jnp.ones((16,), jnp.float32)
        zv = jnp.zeros((16,), jnp.float32)
        for r in range(_CH):
            onesb[r, pl.ds(0, 16)] = ones
        for r in range(8):
            z16[r, pl.ds(0, 16)] = zv
        _zero_shared(zb, accx, sid, 128)

        def zbody16(j, carry):
            pltpu.sync_copy(z16, accd.at[pl.ds(sid * _RPT + j * 8, 8)])
            pltpu.sync_copy(z16, accb.at[pl.ds(sid * _RPT + j * 8, 8)])
            return carry

        lax.fori_loop(0, _RPT // 8, zbody16, 0)
        plsc.subcore_barrier()

        def body(i, carry):
            base = wid * _EW + i * _CH
            pltpu.sync_copy(rowh.at[pl.ds(base, _CH)], rowb)
            pltpu.sync_copy(row2h.at[pl.ds(base, _CH)], rb2)
            pltpu.sync_copy(colh.at[pl.ds(base, _CH)], colb)
            pltpu.async_copy(tab.at[colb], gbuf, sem).wait()
            pltpu.sync_copy(gbuf, accx.at[rb2], add=True)
            pltpu.sync_copy(onesb, accd.at[rowb], add=True)
            pltpu.sync_copy(onesb, accb.at[rb2], add=True)
            return carry

        lax.fori_loop(0, _NCH, body, 0)
        plsc.subcore_barrier()
        pltpu.sync_copy(accx.at[pl.ds(sid * _RPT, _RPT)],
                        xout.at[cid].at[pl.ds(sid * _RPT, _RPT)])
        pltpu.sync_copy(accd.at[pl.ds(sid * _RPT, _RPT)],
                        dout.at[cid].at[pl.ds(sid * _RPT, _RPT)])
        pltpu.sync_copy(accb.at[pl.ds(sid * _RPT, _RPT)],
                        bout.at[cid].at[pl.ds(sid * _RPT, _RPT)])

    return f


def kernel(edge_index, x, struct_x, static_x,
           W_st0, b_st0, W_st1, b_st1, W_sa0, b_sa0, W_sa1, b_sa1,
           W_n0, b_n0, W_n1, b_n1, W_m0, b_m0, W_m1, b_m1,
           W_f0, b_f0, W_f1, b_f1, W_out, b_out):
    row = edge_index[0]
    col = edge_index[1]
    r2 = lambda b: b.reshape(1, -1)

    hs, ha, hn, f2 = _tck1(x, struct_x, static_x, W_st0, W_sa0, W_n0,
                           W_f0, r2(b_f0), W_f1, r2(b_f1))

    row2 = _tck0(row, col)
    xparts, dparts, bparts = _sc_agg_x_deg()(x, row, row2, col)
    t1, t1b, dinv, binv = _tck2(dparts[0, :_N, 0:1], dparts[1, :_N, 0:1],
                                bparts[0, :_N, 0:1], bparts[1, :_N, 0:1],
                                hs, ha, hn)

    p1 = _sc_agg(128)(t1, row, col)
    q1 = _sc_agg(64)(t1b, row, col)
    t2, t2b = _tck3(p1[0, :_N], p1[1, :_N], q1[0, :_N], q1[1, :_N], t1, t1b,
                    xparts[0, :_N], xparts[1, :_N], dinv, binv,
                    W_st1, W_sa1, W_n1, W_m0, r2(b_st0), r2(b_sa0), r2(b_n0))

    p3 = _sc_agg(128)(t2, row, col)
    p4 = _sc_agg(128)(t2b, row, col)
    san, t5 = _tck4(p3[0, :_N], p3[1, :_N], t2, p4[0, :_N], p4[1, :_N], t2b,
                    dinv, r2(b_st1), r2(b_sa1), r2(b_n1), r2(b_m0), W_m1)

    p5 = _sc_agg(64)(t5, row, col)
    out = _tck5(san, f2, p5[0, :_N], p5[1, :_N], t5, dinv, r2(b_m1),
                W_out[0:192, :], W_out[192:256, :], W_out[256:320, :],
                r2(b_out))
    return out


# speedup vs baseline: 24.8793x; 2.1061x over previous
import functools

import jax
import jax.numpy as jnp
from jax import lax
from jax.experimental import pallas as pl
from jax.experimental.pallas import tpu as pltpu
from jax.experimental.pallas import tpu_sc as plsc

_N = 10000
_E = 320000
_R = 2000
_G = _N // _R

_NP = 10112
_RPT = _NP // 16
_EW = _E // 32
_CH = 100
_NCH = _EW // _CH
_ECH = _E // _CH
_TRASH = 10048


def _rowspec(k):
    return pl.BlockSpec((_R, k), lambda i: (i, 0))


def _fullspec(shape):
    nd = len(shape)
    return pl.BlockSpec(shape, lambda i: (0,) * nd)


def _l2n(a):
    n = jnp.sqrt(jnp.sum(a * a, axis=1, keepdims=True))
    return a / jnp.maximum(n, 1e-12)


def _tck0_body(r_ref, c_ref, out_ref):
    r = r_ref[...]
    out_ref[...] = jnp.where(r == c_ref[...], _TRASH, r)


def _tck0(row, col):
    rr = row.reshape(2500, 128)
    cr = col.reshape(2500, 128)
    out = pl.pallas_call(
        _tck0_body,
        grid=(1,),
        in_specs=[pl.BlockSpec((2500, 128), lambda i: (0, 0))] * 2,
        out_specs=pl.BlockSpec((2500, 128), lambda i: (0, 0)),
        out_shape=jax.ShapeDtypeStruct((2500, 128), jnp.int32),
    )(rr, cr)
    return out.reshape(_E)


def _tck1_body(x_ref, sx_ref, ax_ref, W_st0, W_sa0, W_n0, W_f0, b_f0, W_f1, b_f1,
               hs_out, ha_out, hn_out, f2_out):
    x = x_ref[...]
    hs_out[...] = jnp.dot(_l2n(sx_ref[...]), W_st0[...],
                          preferred_element_type=jnp.float32)
    ha_out[...] = jnp.dot(_l2n(ax_ref[...]), W_sa0[...],
                          preferred_element_type=jnp.float32)
    hn_out[...] = jnp.dot(x, W_n0[...], preferred_element_type=jnp.float32)
    f1 = jnp.maximum(jnp.dot(x, W_f0[...], preferred_element_type=jnp.float32)
                     + b_f0[...], 0.0)
    f2_out[...] = jnp.maximum(jnp.dot(f1, W_f1[...],
                                      preferred_element_type=jnp.float32)
                              + b_f1[...], 0.0)


def _tck1(x, struct_x, static_x, W_st0, W_sa0, W_n0, W_f0, b_f0, W_f1, b_f1):
    o64 = jax.ShapeDtypeStruct((_N, 64), jnp.float32)
    return pl.pallas_call(
        _tck1_body,
        grid=(_G,),
        in_specs=[_rowspec(128), _rowspec(64), _rowspec(9),
                  _fullspec((64, 64)), _fullspec((9, 64)), _fullspec((128, 64)),
                  _fullspec((128, 64)), _fullspec((1, 64)),
                  _fullspec((64, 64)), _fullspec((1, 64))],
        out_specs=[_rowspec(64)] * 4,
        out_shape=[o64, o64, o64, o64],
    )(x, struct_x, static_x, W_st0, W_sa0, W_n0, W_f0, b_f0, W_f1, b_f1)


def _tck2_body(d0, d1, b0, b1, hs, ha, hn, t1_out, t1b_out,
               dinv_out, binv_out):
    deg_raw = d0[...] + d1[...]
    deg_b = b0[...] + b1[...]
    dinv = lax.rsqrt(deg_raw + 1.0)
    binv = jnp.where(deg_b > 0, 1.0 / jnp.maximum(deg_b, 1e-12), 0.0)
    t1_out[:, 0:64] = dinv * hs[...]
    t1_out[:, 64:128] = dinv * ha[...]
    t1b_out[...] = dinv * hn[...]
    dinv_out[...] = dinv
    binv_out[...] = binv


def _tck2(deg0, deg1, degb0, degb1, hs, ha, hn):
    o1 = jax.ShapeDtypeStruct((_N, 1), jnp.float32)
    return pl.pallas_call(
        _tck2_body,
        grid=(_G,),
        in_specs=[_rowspec(1)] * 4 + [_rowspec(64)] * 3,
        out_specs=[_rowspec(128), _rowspec(64), _rowspec(1), _rowspec(1)],
        out_shape=[jax.ShapeDtypeStruct((_N, 128), jnp.float32),
                   jax.ShapeDtypeStruct((_N, 64), jnp.float32), o1, o1],
    )(deg0, deg1, degb0, degb1, hs, ha, hn)


def _tck3_body(p1a, p1b, q1a, q1b, t1, t1b, xa, xb,
               dinv_ref, binv_ref,
               W_st1, W_sa1, W_n1, W_m0, b_st0, b_sa0, b_n0,
               t2_out, t2b_out):
    dinv = dinv_ref[...]
    v = dinv * (p1a[...] + p1b[...] + t1[...])
    s1 = jnp.maximum(v[:, 0:64] + b_st0[...], 0.0)
    a1 = jnp.maximum(v[:, 64:128] + b_sa0[...], 0.0)
    n1 = jnp.maximum(dinv * (q1a[...] + q1b[...] + t1b[...]) + b_n0[...], 0.0)
    mxagg = binv_ref[...] * (xa[...] + xb[...])
    mx = _l2n(mxagg)
    t2_out[:, 0:64] = dinv * jnp.dot(s1, W_st1[...],
                                     preferred_element_type=jnp.float32)
    t2_out[:, 64:128] = dinv * jnp.dot(a1, W_sa1[...],
                                       preferred_element_type=jnp.float32)
    t2b_out[:, 0:64] = dinv * jnp.dot(n1, W_n1[...],
                                      preferred_element_type=jnp.float32)
    t2b_out[:, 64:128] = dinv * jnp.dot(mx, W_m0[...],
                                        preferred_element_type=jnp.float32)


def _tck3(p1a, p1b, q1a, q1b, t1, t1b, xa, xb, dinv, binv,
          W_st1, W_sa1, W_n1, W_m0, b_st0, b_sa0, b_n0):
    return pl.pallas_call(
        _tck3_body,
        grid=(_G,),
        in_specs=[_rowspec(128), _rowspec(128), _rowspec(64), _rowspec(64),
                  _rowspec(128), _rowspec(64),
                  _rowspec(128), _rowspec(128),
                  _rowspec(1), _rowspec(1),
                  _fullspec((64, 64)), _fullspec((64, 64)), _fullspec((64, 64)),
                  _fullspec((128, 64)),
                  _fullspec((1, 64)), _fullspec((1, 64)), _fullspec((1, 64))],
        out_specs=[_rowspec(128), _rowspec(128)],
        out_shape=[jax.ShapeDtypeStruct((_N, 128), jnp.float32),
                   jax.ShapeDtypeStruct((_N, 128), jnp.float32)],
    )(p1a, p1b, q1a, q1b, t1, t1b, xa, xb, dinv, binv,
      W_st1, W_sa1, W_n1, W_m0, b_st0, b_sa0, b_n0)


def _tck4_body(p3a, p3b, t2, p4a, p4b, t2b, dinv_ref,
               b_st1, b_sa1, b_n1, b_m0, W_m1, san_out, t5_out):
    dinv = dinv_ref[...]
    v = dinv * (p3a[...] + p3b[...] + t2[...])
    san_out[:, 0:64] = jnp.maximum(v[:, 0:64] + b_st1[...], 0.0)
    san_out[:, 64:128] = jnp.maximum(v[:, 64:128] + b_sa1[...], 0.0)
    vb = dinv * (p4a[...] + p4b[...] + t2b[...])
    san_out[:, 128:192] = jnp.maximum(vb[:, 0:64] + b_n1[...], 0.0)
    m1 = jnp.maximum(vb[:, 64:128] + b_m0[...], 0.0)
    t5_out[...] = dinv * jnp.dot(m1, W_m1[...],
                                 preferred_element_type=jnp.float32)


def _tck4(p3a, p3b, t2, p4a, p4b, t2b, dinv, b_st1, b_sa1, b_n1, b_m0, W_m1):
    return pl.pallas_call(
        _tck4_body,
        grid=(_G,),
        in_specs=[_rowspec(128), _rowspec(128), _rowspec(128),
                  _rowspec(128), _rowspec(128), _rowspec(128), _rowspec(1),
                  _fullspec((1, 64)), _fullspec((1, 64)), _fullspec((1, 64)),
                  _fullspec((1, 64)), _fullspec((64, 64))],
        out_specs=[_rowspec(192), _rowspec(64)],
        out_shape=[jax.ShapeDtypeStruct((_N, 192), jnp.float32),
                   jax.ShapeDtypeStruct((_N, 64), jnp.float32)],
    )(p3a, p3b, t2, p4a, p4b, t2b, dinv, b_st1, b_sa1, b_n1, b_m0, W_m1)


def _tck5_body(san, f2, p5a, p5b, t5, dinv_ref, b_m1,
               Wo_a, Wo_m, Wo_f, b_out, out_ref):
    m2 = jnp.maximum(dinv_ref[...] * (p5a[...] + p5b[...] + t5[...])
                     + b_m1[...], 0.0)
    out_ref[...] = (jnp.dot(san[...], Wo_a[...],
                            preferred_element_type=jnp.float32)
                    + jnp.dot(m2, Wo_m[...],
                              preferred_element_type=jnp.float32)
                    + jnp.dot(f2[...], Wo_f[...],
                              preferred_element_type=jnp.float32)
                    + b_out[...])


def _tck5(san, f2, p5a, p5b, t5, dinv, b_m1, Wo_a, Wo_m, Wo_f, b_out):
    return pl.pallas_call(
        _tck5_body,
        grid=(_G,),
        in_specs=[_rowspec(192), _rowspec(64), _rowspec(64), _rowspec(64),
                  _rowspec(64), _rowspec(1),
                  _fullspec((1, 64)), _fullspec((192, 40)),
                  _fullspec((64, 40)), _fullspec((64, 40)), _fullspec((1, 40))],
        out_specs=_rowspec(40),
        out_shape=jax.ShapeDtypeStruct((_N, 40), jnp.float32),
    )(san, f2, p5a, p5b, t5, dinv, b_m1, Wo_a, Wo_m, Wo_f, b_out)


def _zero_shared(zb, acc, sid, width, semz):
    zv = jnp.zeros((16,), jnp.float32)
    for r in range(8):
        for c2 in range(width // 16):
            zb[r, pl.ds(c2 * 16, 16)] = zv

    def zbody(j, carry):
        pltpu.async_copy(zb, acc.at[pl.ds(sid * _RPT + j * 8, 8)], semz)
        return carry

    lax.fori_loop(0, _RPT // 8, zbody, 0)

    def zdrain(j, carry):
        pltpu.make_async_copy(zb, acc.at[pl.ds(sid * _RPT + j * 8, 8)],
                              semz).wait()
        return carry

    lax.fori_loop(0, _RPT // 8, zdrain, 0)


def _sc_agg(width):
    mesh = plsc.VectorSubcoreMesh(core_axis_name="c", subcore_axis_name="s")

    @functools.partial(
        pl.kernel, mesh=mesh,
        out_type=jax.ShapeDtypeStruct((2, _NP, width), jnp.float32),
        compiler_params=pltpu.CompilerParams(use_tc_tiling_on_sc=False),
        scratch_types=[
            pltpu.VMEM((_NCH, _CH), jnp.int32),
            pltpu.VMEM((_NCH, _CH), jnp.int32),
            pltpu.VMEM((_CH, width), jnp.float32),
            pltpu.VMEM((_CH, width), jnp.float32),
            pltpu.VMEM((8, width), jnp.float32),
            pltpu.VMEM_SHARED((_NP, width), jnp.float32),
            pltpu.SemaphoreType.DMA,
            pltpu.SemaphoreType.DMA,
            pltpu.SemaphoreType.DMA,
        ])
    def f(tab, rowh, colh, out, rowb, colb, g0, g1, zb, acc, sema, semb, semz):
        cid = lax.axis_index("c")
        sid = lax.axis_index("s")
        wid = sid * 2 + cid
        pltpu.sync_copy(rowh.at[pl.ds(wid * _NCH, _NCH)], rowb)
        pltpu.sync_copy(colh.at[pl.ds(wid * _NCH, _NCH)], colb)
        _zero_shared(zb, acc, sid, width, semz)
        plsc.subcore_barrier()

        pltpu.async_copy(tab.at[colb.at[0]], g0, sema)

        def body(j, carry):
            i0 = 2 * j
            pltpu.make_async_copy(tab.at[colb.at[i0]], g0, sema).wait()
            pltpu.async_copy(tab.at[colb.at[i0 + 1]], g1, semb)
            pltpu.sync_copy(g0, acc.at[rowb.at[i0]], add=True)
            pltpu.make_async_copy(tab.at[colb.at[i0 + 1]], g1, semb).wait()

            @pl.when(j < _NCH // 2 - 1)
            def _():
                pltpu.async_copy(tab.at[colb.at[i0 + 2]], g0, sema)

            pltpu.sync_copy(g1, acc.at[rowb.at[i0 + 1]], add=True)
            return carry

        lax.fori_loop(0, _NCH // 2, body, 0)
        plsc.subcore_barrier()
        pltpu.sync_copy(acc.at[pl.ds(sid * _RPT, _RPT)],
                        out.at[cid].at[pl.ds(sid * _RPT, _RPT)])

    return f


def _sc_deg():
    mesh = plsc.VectorSubcoreMesh(core_axis_name="c", subcore_axis_name="s")

    @functools.partial(
        pl.kernel, mesh=mesh,
        out_type=[jax.ShapeDtypeStruct((2, _NP, 16), jnp.float32),
                  jax.ShapeDtypeStruct((2, _NP, 16), jnp.float32)],
        compiler_params=pltpu.CompilerParams(use_tc_tiling_on_sc=False),
        scratch_types=[
            pltpu.VMEM((_NCH, _CH), jnp.int32),
            pltpu.VMEM((_NCH, _CH), jnp.int32),
            pltpu.VMEM((_CH, 16), jnp.float32),
            pltpu.VMEM((8, 16), jnp.float32),
            pltpu.VMEM_SHARED((_NP, 16), jnp.float32),
            pltpu.VMEM_SHARED((_NP, 16), jnp.float32),
            pltpu.SemaphoreType.DMA,
        ])
    def f(rowh, row2h, dout, bout, rowb, rb2, onesb, z16, accd, accb, semz):
        cid = lax.axis_index("c")
        sid = lax.axis_index("s")
        wid = sid * 2 + cid
        ones = jnp.ones((16,), jnp.float32)
        for r in range(_CH):
            onesb[r, pl.ds(0, 16)] = ones
        pltpu.sync_copy(rowh.at[pl.ds(wid * _NCH, _NCH)], rowb)
        pltpu.sync_copy(row2h.at[pl.ds(wid * _NCH, _NCH)], rb2)
        _zero_shared(z16, accd, sid, 16, semz)
        _zero_shared(z16, accb, sid, 16, semz)
        plsc.subcore_barrier()

        def body(i, carry):
            pltpu.sync_copy(onesb, accd.at[rowb.at[i]], add=True)
            pltpu.sync_copy(onesb, accb.at[rb2.at[i]], add=True)
            return carry

        lax.fori_loop(0, _NCH, body, 0)
        plsc.subcore_barrier()
        pltpu.sync_copy(accd.at[pl.ds(sid * _RPT, _RPT)],
                        dout.at[cid].at[pl.ds(sid * _RPT, _RPT)])
        pltpu.sync_copy(accb.at[pl.ds(sid * _RPT, _RPT)],
                        bout.at[cid].at[pl.ds(sid * _RPT, _RPT)])

    return f


def kernel(edge_index, x, struct_x, static_x,
           W_st0, b_st0, W_st1, b_st1, W_sa0, b_sa0, W_sa1, b_sa1,
           W_n0, b_n0, W_n1, b_n1, W_m0, b_m0, W_m1, b_m1,
           W_f0, b_f0, W_f1, b_f1, W_out, b_out):
    row = edge_index[0]
    col = edge_index[1]
    rowr = row.reshape(_ECH, _CH)
    colr = col.reshape(_ECH, _CH)
    r2 = lambda b: b.reshape(1, -1)

    hs, ha, hn, f2 = _tck1(x, struct_x, static_x, W_st0, W_sa0, W_n0,
                           W_f0, r2(b_f0), W_f1, r2(b_f1))

    row2r = _tck0(row, col).reshape(_ECH, _CH)
    xparts = _sc_agg(128)(x, row2r, colr)
    dparts, bparts = _sc_deg()(rowr, row2r)
    t1, t1b, dinv, binv = _tck2(dparts[0, :_N, 0:1], dparts[1, :_N, 0:1],
                                bparts[0, :_N, 0:1], bparts[1, :_N, 0:1],
                                hs, ha, hn)

    p1 = _sc_agg(128)(t1, rowr, colr)
    q1 = _sc_agg(64)(t1b, rowr, colr)
    t2, t2b = _tck3(p1[0, :_N], p1[1, :_N], q1[0, :_N], q1[1, :_N], t1, t1b,
                    xparts[0, :_N], xparts[1, :_N], dinv, binv,
                    W_st1, W_sa1, W_n1, W_m0, r2(b_st0), r2(b_sa0), r2(b_n0))

    p3 = _sc_agg(128)(t2, rowr, colr)
    p4 = _sc_agg(128)(t2b, rowr, colr)
    san, t5 = _tck4(p3[0, :_N], p3[1, :_N], t2, p4[0, :_N], p4[1, :_N], t2b,
                    dinv, r2(b_st1), r2(b_sa1), r2(b_n1), r2(b_m0), W_m1)

    p5 = _sc_agg(64)(t5, rowr, colr)
    out = _tck5(san, f2, p5[0, :_N], p5[1, :_N], t5, dinv, r2(b_m1),
                W_out[0:192, :], W_out[192:256, :], W_out[256:320, :],
                r2(b_out))
    return out


# speedup vs baseline: 25.3319x; 1.0182x over previous
import functools

import jax
import jax.numpy as jnp
from jax import lax
from jax.experimental import pallas as pl
from jax.experimental.pallas import tpu as pltpu
from jax.experimental.pallas import tpu_sc as plsc

_N = 10000
_E = 320000
_R = 2000
_G = _N // _R

_NP = 10112
_RPT = _NP // 16
_EW = _E // 32
_CH = 100
_NCH = _EW // _CH
_ECH = _E // _CH
_TRASH = 10048


def _rowspec(k):
    return pl.BlockSpec((_R, k), lambda i: (i, 0))


def _fullspec(shape):
    nd = len(shape)
    return pl.BlockSpec(shape, lambda i: (0,) * nd)


def _l2n(a):
    n = jnp.sqrt(jnp.sum(a * a, axis=1, keepdims=True))
    return a / jnp.maximum(n, 1e-12)


def _tck0_body(r_ref, c_ref, out_ref):
    r = r_ref[...]
    out_ref[...] = jnp.where(r == c_ref[...], _TRASH, r)


def _tck0(row, col):
    rr = row.reshape(2500, 128)
    cr = col.reshape(2500, 128)
    out = pl.pallas_call(
        _tck0_body,
        grid=(1,),
        in_specs=[pl.BlockSpec((2500, 128), lambda i: (0, 0))] * 2,
        out_specs=pl.BlockSpec((2500, 128), lambda i: (0, 0)),
        out_shape=jax.ShapeDtypeStruct((2500, 128), jnp.int32),
    )(rr, cr)
    return out.reshape(_E)


def _tck1_body(x_ref, sx_ref, ax_ref, W_st0, W_sa0, W_n0, W_f0, b_f0, W_f1, b_f1,
               hs_out, ha_out, hn_out, f2_out):
    x = x_ref[...]
    hs_out[...] = jnp.dot(_l2n(sx_ref[...]), W_st0[...],
                          preferred_element_type=jnp.float32)
    ha_out[...] = jnp.dot(_l2n(ax_ref[...]), W_sa0[...],
                          preferred_element_type=jnp.float32)
    hn_out[...] = jnp.dot(x, W_n0[...], preferred_element_type=jnp.float32)
    f1 = jnp.maximum(jnp.dot(x, W_f0[...], preferred_element_type=jnp.float32)
                     + b_f0[...], 0.0)
    f2_out[...] = jnp.maximum(jnp.dot(f1, W_f1[...],
                                      preferred_element_type=jnp.float32)
                              + b_f1[...], 0.0)


def _tck1(x, struct_x, static_x, W_st0, W_sa0, W_n0, W_f0, b_f0, W_f1, b_f1):
    o64 = jax.ShapeDtypeStruct((_N, 64), jnp.float32)
    return pl.pallas_call(
        _tck1_body,
        grid=(_G,),
        in_specs=[_rowspec(128), _rowspec(64), _rowspec(9),
                  _fullspec((64, 64)), _fullspec((9, 64)), _fullspec((128, 64)),
                  _fullspec((128, 64)), _fullspec((1, 64)),
                  _fullspec((64, 64)), _fullspec((1, 64))],
        out_specs=[_rowspec(64)] * 4,
        out_shape=[o64, o64, o64, o64],
    )(x, struct_x, static_x, W_st0, W_sa0, W_n0, W_f0, b_f0, W_f1, b_f1)


def _tck2_body(d0, d1, b0, b1, hs, ha, hn, t1_out, t1b_out,
               dinv_out, binv_out):
    deg_raw = d0[...] + d1[...]
    deg_b = b0[...] + b1[...]
    dinv = lax.rsqrt(deg_raw + 1.0)
    binv = jnp.where(deg_b > 0, 1.0 / jnp.maximum(deg_b, 1e-12), 0.0)
    t1_out[:, 0:64] = dinv * hs[...]
    t1_out[:, 64:128] = dinv * ha[...]
    t1b_out[...] = dinv * hn[...]
    dinv_out[...] = dinv
    binv_out[...] = binv


def _tck2(deg0, deg1, degb0, degb1, hs, ha, hn):
    o1 = jax.ShapeDtypeStruct((_N, 1), jnp.float32)
    return pl.pallas_call(
        _tck2_body,
        grid=(_G,),
        in_specs=[_rowspec(1)] * 4 + [_rowspec(64)] * 3,
        out_specs=[_rowspec(128), _rowspec(64), _rowspec(1), _rowspec(1)],
        out_shape=[jax.ShapeDtypeStruct((_N, 128), jnp.float32),
                   jax.ShapeDtypeStruct((_N, 64), jnp.float32), o1, o1],
    )(deg0, deg1, degb0, degb1, hs, ha, hn)


def _tck3_body(p1a, p1b, q1a, q1b, t1, t1b, xa, xb,
               dinv_ref, binv_ref,
               W_st1, W_sa1, W_n1, W_m0, b_st0, b_sa0, b_n0,
               t2_out, t2b_out):
    dinv = dinv_ref[...]
    v = dinv * (p1a[...] + p1b[...] + t1[...])
    s1 = jnp.maximum(v[:, 0:64] + b_st0[...], 0.0)
    a1 = jnp.maximum(v[:, 64:128] + b_sa0[...], 0.0)
    n1 = jnp.maximum(dinv * (q1a[...] + q1b[...] + t1b[...]) + b_n0[...], 0.0)
    mxagg = binv_ref[...] * (xa[...] + xb[...])
    mx = _l2n(mxagg)
    t2_out[:, 0:64] = dinv * jnp.dot(s1, W_st1[...],
                                     preferred_element_type=jnp.float32)
    t2_out[:, 64:128] = dinv * jnp.dot(a1, W_sa1[...],
                                       preferred_element_type=jnp.float32)
    t2b_out[:, 0:64] = dinv * jnp.dot(n1, W_n1[...],
                                      preferred_element_type=jnp.float32)
    t2b_out[:, 64:128] = dinv * jnp.dot(mx, W_m0[...],
                                        preferred_element_type=jnp.float32)


def _tck3(p1a, p1b, q1a, q1b, t1, t1b, xa, xb, dinv, binv,
          W_st1, W_sa1, W_n1, W_m0, b_st0, b_sa0, b_n0):
    return pl.pallas_call(
        _tck3_body,
        grid=(_G,),
        in_specs=[_rowspec(128), _rowspec(128), _rowspec(64), _rowspec(64),
                  _rowspec(128), _rowspec(64),
                  _rowspec(128), _rowspec(128),
                  _rowspec(1), _rowspec(1),
                  _fullspec((64, 64)), _fullspec((64, 64)), _fullspec((64, 64)),
                  _fullspec((128, 64)),
                  _fullspec((1, 64)), _fullspec((1, 64)), _fullspec((1, 64))],
        out_specs=[_rowspec(128), _rowspec(128)],
        out_shape=[jax.ShapeDtypeStruct((_N, 128), jnp.float32),
                   jax.ShapeDtypeStruct((_N, 128), jnp.float32)],
    )(p1a, p1b, q1a, q1b, t1, t1b, xa, xb, dinv, binv,
      W_st1, W_sa1, W_n1, W_m0, b_st0, b_sa0, b_n0)


def _tck4_body(p3a, p3b, t2, p4a, p4b, t2b, dinv_ref,
               b_st1, b_sa1, b_n1, b_m0, W_m1, san_out, t5_out):
    dinv = dinv_ref[...]
    v = dinv * (p3a[...] + p3b[...] + t2[...])
    san_out[:, 0:64] = jnp.maximum(v[:, 0:64] + b_st1[...], 0.0)
    san_out[:, 64:128] = jnp.maximum(v[:, 64:128] + b_sa1[...], 0.0)
    vb = dinv * (p4a[...] + p4b[...] + t2b[...])
    san_out[:, 128:192] = jnp.maximum(vb[:, 0:64] + b_n1[...], 0.0)
    m1 = jnp.maximum(vb[:, 64:128] + b_m0[...], 0.0)
    t5_out[...] = dinv * jnp.dot(m1, W_m1[...],
                                 preferred_element_type=jnp.float32)


def _tck4(p3a, p3b, t2, p4a, p4b, t2b, dinv, b_st1, b_sa1, b_n1, b_m0, W_m1):
    return pl.pallas_call(
        _tck4_body,
        grid=(_G,),
        in_specs=[_rowspec(128), _rowspec(128), _rowspec(128),
                  _rowspec(128), _rowspec(128), _rowspec(128), _rowspec(1),
                  _fullspec((1, 64)), _fullspec((1, 64)), _fullspec((1, 64)),
                  _fullspec((1, 64)), _fullspec((64, 64))],
        out_specs=[_rowspec(192), _rowspec(64)],
        out_shape=[jax.ShapeDtypeStruct((_N, 192), jnp.float32),
                   jax.ShapeDtypeStruct((_N, 64), jnp.float32)],
    )(p3a, p3b, t2, p4a, p4b, t2b, dinv, b_st1, b_sa1, b_n1, b_m0, W_m1)


def _tck5_body(san, f2, p5a, p5b, t5, dinv_ref, b_m1,
               Wo_a, Wo_m, Wo_f, b_out, out_ref):
    m2 = jnp.maximum(dinv_ref[...] * (p5a[...] + p5b[...] + t5[...])
                     + b_m1[...], 0.0)
    out_ref[...] = (jnp.dot(san[...], Wo_a[...],
                            preferred_element_type=jnp.float32)
                    + jnp.dot(m2, Wo_m[...],
                              preferred_element_type=jnp.float32)
                    + jnp.dot(f2[...], Wo_f[...],
                              preferred_element_type=jnp.float32)
                    + b_out[...])


def _tck5(san, f2, p5a, p5b, t5, dinv, b_m1, Wo_a, Wo_m, Wo_f, b_out):
    return pl.pallas_call(
        _tck5_body,
        grid=(_G,),
        in_specs=[_rowspec(192), _rowspec(64), _rowspec(64), _rowspec(64),
                  _rowspec(64), _rowspec(1),
                  _fullspec((1, 64)), _fullspec((192, 40)),
                  _fullspec((64, 40)), _fullspec((64, 40)), _fullspec((1, 40))],
        out_specs=_rowspec(40),
        out_shape=jax.ShapeDtypeStruct((_N, 40), jnp.float32),
    )(san, f2, p5a, p5b, t5, dinv, b_m1, Wo_a, Wo_m, Wo_f, b_out)


def _zero_shared(zb, acc, sid, width, semz):
    zv = jnp.zeros((16,), jnp.float32)
    for r in range(8):
        for c2 in range(width // 16):
            zb[r, pl.ds(c2 * 16, 16)] = zv

    def zbody(j, carry):
        pltpu.async_copy(zb, acc.at[pl.ds(sid * _RPT + j * 8, 8)], semz)
        return carry

    lax.fori_loop(0, _RPT // 8, zbody, 0)

    def zdrain(j, carry):
        pltpu.make_async_copy(zb, acc.at[pl.ds(sid * _RPT + j * 8, 8)],
                              semz).wait()
        return carry

    lax.fori_loop(0, _RPT // 8, zdrain, 0)


def _sc_agg(width, ch=_CH):
    nch = _EW // ch
    mesh = plsc.VectorSubcoreMesh(core_axis_name="c", subcore_axis_name="s")

    @functools.partial(
        pl.kernel, mesh=mesh,
        out_type=jax.ShapeDtypeStruct((2, _NP, width), jnp.float32),
        compiler_params=pltpu.CompilerParams(use_tc_tiling_on_sc=False),
        scratch_types=[
            pltpu.VMEM((nch, ch), jnp.int32),
            pltpu.VMEM((nch, ch), jnp.int32),
            pltpu.VMEM((ch, width), jnp.float32),
            pltpu.VMEM((ch, width), jnp.float32),
            pltpu.VMEM((8, width), jnp.float32),
            pltpu.VMEM_SHARED((_NP, width), jnp.float32),
            pltpu.SemaphoreType.DMA,
            pltpu.SemaphoreType.DMA,
            pltpu.SemaphoreType.DMA,
        ])
    def f(tab, rowh, colh, out, rowb, colb, g0, g1, zb, acc, sema, semb, semz):
        cid = lax.axis_index("c")
        sid = lax.axis_index("s")
        wid = sid * 2 + cid
        pltpu.sync_copy(rowh.at[pl.ds(wid * nch, nch)], rowb)
        pltpu.sync_copy(colh.at[pl.ds(wid * nch, nch)], colb)
        _zero_shared(zb, acc, sid, width, semz)
        plsc.subcore_barrier()

        pltpu.async_copy(tab.at[colb.at[0]], g0, sema)

        def body(j, carry):
            i0 = 2 * j
            pltpu.make_async_copy(tab.at[colb.at[i0]], g0, sema).wait()
            pltpu.async_copy(tab.at[colb.at[i0 + 1]], g1, semb)
            pltpu.sync_copy(g0, acc.at[rowb.at[i0]], add=True)
            pltpu.make_async_copy(tab.at[colb.at[i0 + 1]], g1, semb).wait()

            @pl.when(j < nch // 2 - 1)
            def _():
                pltpu.async_copy(tab.at[colb.at[i0 + 2]], g0, sema)

            pltpu.sync_copy(g1, acc.at[rowb.at[i0 + 1]], add=True)
            return carry

        lax.fori_loop(0, nch // 2, body, 0)
        plsc.subcore_barrier()
        pltpu.sync_copy(acc.at[pl.ds(sid * _RPT, _RPT)],
                        out.at[cid].at[pl.ds(sid * _RPT, _RPT)])

    return f


def _sc_agg2():
    mesh = plsc.VectorSubcoreMesh(core_axis_name="c", subcore_axis_name="s")

    @functools.partial(
        pl.kernel, mesh=mesh,
        out_type=[jax.ShapeDtypeStruct((2, _NP, 128), jnp.float32),
                  jax.ShapeDtypeStruct((2, _NP, 128), jnp.float32)],
        compiler_params=pltpu.CompilerParams(use_tc_tiling_on_sc=False),
        scratch_types=[
            pltpu.VMEM((_NCH, _CH), jnp.int32),
            pltpu.VMEM((_NCH, _CH), jnp.int32),
            pltpu.VMEM((_CH, 128), jnp.float32),
            pltpu.VMEM((_CH, 128), jnp.float32),
            pltpu.VMEM((8, 128), jnp.float32),
            pltpu.VMEM_SHARED((_NP, 128), jnp.float32),
            pltpu.SemaphoreType.DMA,
            pltpu.SemaphoreType.DMA,
            pltpu.SemaphoreType.DMA,
        ])
    def f(taba, tabb, rowh, colh, outa, outb,
          rowb, colb, g0, g1, zb, acc, sema, semb, semz):
        cid = lax.axis_index("c")
        sid = lax.axis_index("s")
        wid = sid * 2 + cid
        pltpu.sync_copy(rowh.at[pl.ds(wid * _NCH, _NCH)], rowb)
        pltpu.sync_copy(colh.at[pl.ds(wid * _NCH, _NCH)], colb)

        def one_pass(tab, out):
            _zero_shared(zb, acc, sid, 128, semz)
            plsc.subcore_barrier()
            pltpu.async_copy(tab.at[colb.at[0]], g0, sema)

            def body(j, carry):
                i0 = 2 * j
                pltpu.make_async_copy(tab.at[colb.at[i0]], g0, sema).wait()
                pltpu.async_copy(tab.at[colb.at[i0 + 1]], g1, semb)
                pltpu.sync_copy(g0, acc.at[rowb.at[i0]], add=True)
                pltpu.make_async_copy(tab.at[colb.at[i0 + 1]], g1, semb).wait()

                @pl.when(j < _NCH // 2 - 1)
                def _():
                    pltpu.async_copy(tab.at[colb.at[i0 + 2]], g0, sema)

                pltpu.sync_copy(g1, acc.at[rowb.at[i0 + 1]], add=True)
                return carry

            lax.fori_loop(0, _NCH // 2, body, 0)
            plsc.subcore_barrier()
            pltpu.sync_copy(acc.at[pl.ds(sid * _RPT, _RPT)],
                            out.at[cid].at[pl.ds(sid * _RPT, _RPT)])

        one_pass(taba, outa)
        plsc.subcore_barrier()
        one_pass(tabb, outb)

    return f


def _sc_deg():
    mesh = plsc.VectorSubcoreMesh(core_axis_name="c", subcore_axis_name="s")

    @functools.partial(
        pl.kernel, mesh=mesh,
        out_type=[jax.ShapeDtypeStruct((2, _NP, 16), jnp.float32),
                  jax.ShapeDtypeStruct((2, _NP, 16), jnp.float32)],
        compiler_params=pltpu.CompilerParams(use_tc_tiling_on_sc=False),
        scratch_types=[
            pltpu.VMEM((_NCH, _CH), jnp.int32),
            pltpu.VMEM((_NCH, _CH), jnp.int32),
            pltpu.VMEM((_CH, 16), jnp.float32),
            pltpu.VMEM((8, 16), jnp.float32),
            pltpu.VMEM_SHARED((_NP, 16), jnp.float32),
            pltpu.VMEM_SHARED((_NP, 16), jnp.float32),
            pltpu.SemaphoreType.DMA,
        ])
    def f(rowh, row2h, dout, bout, rowb, rb2, onesb, z16, accd, accb, semz):
        cid = lax.axis_index("c")
        sid = lax.axis_index("s")
        wid = sid * 2 + cid
        ones = jnp.ones((16,), jnp.float32)
        for r in range(_CH):
            onesb[r, pl.ds(0, 16)] = ones
        pltpu.sync_copy(rowh.at[pl.ds(wid * _NCH, _NCH)], rowb)
        pltpu.sync_copy(row2h.at[pl.ds(wid * _NCH, _NCH)], rb2)
        _zero_shared(z16, accd, sid, 16, semz)
        _zero_shared(z16, accb, sid, 16, semz)
        plsc.subcore_barrier()

        def body(i, carry):
            pltpu.async_copy(onesb, accd.at[rowb.at[i]], semz, add=True)
            pltpu.async_copy(onesb, accb.at[rb2.at[i]], semz, add=True)
            return carry

        def drain(i, carry):
            pltpu.make_async_copy(onesb, accd.at[rowb.at[i]], semz).wait()
            pltpu.make_async_copy(onesb, accb.at[rb2.at[i]], semz).wait()
            return carry

        lax.fori_loop(0, _NCH, body, 0)
        lax.fori_loop(0, _NCH, drain, 0)
        plsc.subcore_barrier()
        pltpu.sync_copy(accd.at[pl.ds(sid * _RPT, _RPT)],
                        dout.at[cid].at[pl.ds(sid * _RPT, _RPT)])
        pltpu.sync_copy(accb.at[pl.ds(sid * _RPT, _RPT)],
                        bout.at[cid].at[pl.ds(sid * _RPT, _RPT)])

    return f


def kernel(edge_index, x, struct_x, static_x,
           W_st0, b_st0, W_st1, b_st1, W_sa0, b_sa0, W_sa1, b_sa1,
           W_n0, b_n0, W_n1, b_n1, W_m0, b_m0, W_m1, b_m1,
           W_f0, b_f0, W_f1, b_f1, W_out, b_out):
    row = edge_index[0]
    col = edge_index[1]
    rowr = row.reshape(_ECH, _CH)
    colr = col.reshape(_ECH, _CH)
    r2 = lambda b: b.reshape(1, -1)

    hs, ha, hn, f2 = _tck1(x, struct_x, static_x, W_st0, W_sa0, W_n0,
                           W_f0, r2(b_f0), W_f1, r2(b_f1))

    row2r = _tck0(row, col).reshape(_ECH, _CH)
    xparts = _sc_agg(128)(x, row2r, colr)
    dparts, bparts = _sc_deg()(rowr, row2r)
    t1, t1b, dinv, binv = _tck2(dparts[0, :_N, 0:1], dparts[1, :_N, 0:1],
                                bparts[0, :_N, 0:1], bparts[1, :_N, 0:1],
                                hs, ha, hn)

    p1 = _sc_agg(128)(t1, rowr, colr)
    q1 = _sc_agg(64, 125)(t1b, row.reshape(_E // 125, 125),
                          col.reshape(_E // 125, 125))
    t2, t2b = _tck3(p1[0, :_N], p1[1, :_N], q1[0, :_N], q1[1, :_N], t1, t1b,
                    xparts[0, :_N], xparts[1, :_N], dinv, binv,
                    W_st1, W_sa1, W_n1, W_m0, r2(b_st0), r2(b_sa0), r2(b_n0))

    p3, p4 = _sc_agg2()(t2, t2b, rowr, colr)
    san, t5 = _tck4(p3[0, :_N], p3[1, :_N], t2, p4[0, :_N], p4[1, :_N], t2b,
                    dinv, r2(b_st1), r2(b_sa1), r2(b_n1), r2(b_m0), W_m1)

    p5 = _sc_agg(64, 125)(t5, row.reshape(_E // 125, 125),
                          col.reshape(_E // 125, 125))
    out = _tck5(san, f2, p5[0, :_N], p5[1, :_N], t5, dinv, r2(b_m1),
                W_out[0:192, :], W_out[192:256, :], W_out[256:320, :],
                r2(b_out))
    return out
